# baseline (speedup 1.0000x reference)
import jax
import jax.numpy as jnp
from jax.experimental import pallas as pl
from jax.experimental.pallas import tpu as pltpu

_EMBED = 768
_FFN = 3072
_NEXP = 8
_NT = 2
_TF = _FFN // _NT
_KH = _EMBED // 2


def _moe_body(x_ref, wr_ref, w1a_ref, w1b_ref, b1_ref, w2a_ref, w2b_ref,
              b2_ref, out_ref, w_ref, xs_ref, acc_ref):
    e = pl.program_id(0)
    t = pl.program_id(1)

    @pl.when(jnp.logical_and(e == 0, t == 0))
    def _router():
        xv = x_ref[:, 0, :]
        xs_ref[...] = xv
        logits = jnp.dot(xv, wr_ref[...], preferred_element_type=jnp.float32)
        lane = jax.lax.broadcasted_iota(jnp.int32, logits.shape, 1)
        m1 = jnp.max(logits, axis=-1, keepdims=True)
        pos1 = jnp.min(jnp.where(logits == m1, lane, _NEXP), axis=-1,
                       keepdims=True)
        oh1 = lane == pos1
        l2 = jnp.where(oh1, -jnp.inf, logits)
        m2 = jnp.max(l2, axis=-1, keepdims=True)
        pos2 = jnp.min(jnp.where(l2 == m2, lane, _NEXP), axis=-1,
                       keepdims=True)
        oh2 = lane == pos2
        wa = 1.0 / (1.0 + jnp.exp(m2 - m1))
        w_ref[...] = jnp.where(oh1, wa, 0.0) + jnp.where(oh2, 1.0 - wa, 0.0)
        acc_ref[...] = jnp.zeros_like(acc_ref)

    lane = jax.lax.broadcasted_iota(jnp.int32, (xs_ref.shape[0], _NEXP), 1)
    wcol = jnp.sum(jnp.where(lane == e, w_ref[...], 0.0), axis=-1,
                   keepdims=True)
    h = (jnp.dot(xs_ref[:, :_KH], w1a_ref[0],
                 preferred_element_type=jnp.float32)
         + jnp.dot(xs_ref[:, _KH:], w1b_ref[0],
                   preferred_element_type=jnp.float32))
    h = h + b1_ref[pl.ds(e, 1), pl.ds(pl.multiple_of(t * _TF, 128), _TF)]
    h = 0.5 * h * (1.0 + jax.lax.erf(h * 0.7071067811865476))
    part = (jnp.dot(h[:, :_TF // 2], w2a_ref[0],
                    preferred_element_type=jnp.float32)
            + jnp.dot(h[:, _TF // 2:], w2b_ref[0],
                      preferred_element_type=jnp.float32))
    bias2 = jnp.where(t == 0, 1.0, 0.0) * b2_ref[pl.ds(e, 1), :]
    acc_ref[...] += wcol * (part + bias2)

    @pl.when(jnp.logical_and(e == _NEXP - 1, t == _NT - 1))
    def _writeback():
        out_ref[:, 0, :] = acc_ref[...]


def kernel(x, Wr, W1, B1, W2, B2):
    B, T, C = x.shape
    n_tok = B * T
    out = pl.pallas_call(
        _moe_body,
        grid=(_NEXP, _NT),
        in_specs=[
            pl.BlockSpec((B, T, _EMBED), lambda e, t: (0, 0, 0)),
            pl.BlockSpec((_EMBED, _NEXP), lambda e, t: (0, 0)),
            pl.BlockSpec((1, _KH, _TF), lambda e, t: (e, 0, t)),
            pl.BlockSpec((1, _KH, _TF), lambda e, t: (e, 1, t)),
            pl.BlockSpec((_NEXP, _FFN), lambda e, t: (0, 0)),
            pl.BlockSpec((1, _TF // 2, _EMBED), lambda e, t: (e, 2 * t, 0)),
            pl.BlockSpec((1, _TF // 2, _EMBED),
                         lambda e, t: (e, 2 * t + 1, 0)),
            pl.BlockSpec((_NEXP, _EMBED), lambda e, t: (0, 0)),
        ],
        out_specs=pl.BlockSpec((B, T, _EMBED), lambda e, t: (0, 0, 0)),
        out_shape=jax.ShapeDtypeStruct((B, T, _EMBED), x.dtype),
        scratch_shapes=[
            pltpu.VMEM((n_tok, _NEXP), jnp.float32),
            pltpu.VMEM((n_tok, _EMBED), jnp.float32),
            pltpu.VMEM((n_tok, _EMBED), jnp.float32),
        ],
        compiler_params=pltpu.CompilerParams(
            dimension_semantics=("arbitrary", "arbitrary"),
        ),
    )(x, Wr, W1, W1, B1, W2, W2, B2)
    return out


# speedup vs baseline: 1.0391x; 1.0391x over previous
import jax
import jax.numpy as jnp
from jax.experimental import pallas as pl
from jax.experimental.pallas import tpu as pltpu

_EMBED = 768
_FFN = 3072
_NEXP = 8
_NT = 2
_TF = _FFN // _NT


def _moe_body(x_ref, wr_ref, w1_ref, b1_ref, w2_ref, b2_ref, out_ref,
              w_ref, xs_ref, acc_ref):
    e = pl.program_id(0)
    t = pl.program_id(1)

    @pl.when(jnp.logical_and(e == 0, t == 0))
    def _router():
        xv = x_ref[:, 0, :]
        xs_ref[...] = xv
        logits = jnp.dot(xv, wr_ref[...], preferred_element_type=jnp.float32)
        lane = jax.lax.broadcasted_iota(jnp.int32, logits.shape, 1)
        m1 = jnp.max(logits, axis=-1, keepdims=True)
        pos1 = jnp.min(jnp.where(logits == m1, lane, _NEXP), axis=-1,
                       keepdims=True)
        oh1 = lane == pos1
        l2 = jnp.where(oh1, -jnp.inf, logits)
        m2 = jnp.max(l2, axis=-1, keepdims=True)
        pos2 = jnp.min(jnp.where(l2 == m2, lane, _NEXP), axis=-1,
                       keepdims=True)
        oh2 = lane == pos2
        wa = 1.0 / (1.0 + jnp.exp(m2 - m1))
        w_ref[...] = jnp.where(oh1, wa, 0.0) + jnp.where(oh2, 1.0 - wa, 0.0)
        acc_ref[...] = jnp.zeros_like(acc_ref)

    lane = jax.lax.broadcasted_iota(jnp.int32, (xs_ref.shape[0], _NEXP), 1)
    wcol = jnp.sum(jnp.where(lane == e, w_ref[...], 0.0), axis=-1,
                   keepdims=True)
    h = jnp.dot(xs_ref[...], w1_ref[0], preferred_element_type=jnp.float32)
    h = h + b1_ref[pl.ds(e, 1), pl.ds(pl.multiple_of(t * _TF, 128), _TF)]
    h = 0.5 * h * (1.0 + jax.lax.erf(h * 0.7071067811865476))
    part = jnp.dot(h, w2_ref[0], preferred_element_type=jnp.float32)
    bias2 = jnp.where(t == 0, 1.0, 0.0) * b2_ref[pl.ds(e, 1), :]
    acc_ref[...] += wcol * (part + bias2)

    @pl.when(jnp.logical_and(e == _NEXP - 1, t == _NT - 1))
    def _writeback():
        out_ref[:, 0, :] = acc_ref[...]


def kernel(x, Wr, W1, B1, W2, B2):
    B, T, C = x.shape
    n_tok = B * T
    out = pl.pallas_call(
        _moe_body,
        grid=(_NEXP, _NT),
        in_specs=[
            pl.BlockSpec((B, T, _EMBED), lambda e, t: (0, 0, 0)),
            pl.BlockSpec((_EMBED, _NEXP), lambda e, t: (0, 0)),
            pl.BlockSpec((1, _EMBED, _TF), lambda e, t: (e, 0, t)),
            pl.BlockSpec((_NEXP, _FFN), lambda e, t: (0, 0)),
            pl.BlockSpec((1, _TF, _EMBED), lambda e, t: (e, t, 0)),
            pl.BlockSpec((_NEXP, _EMBED), lambda e, t: (0, 0)),
        ],
        out_specs=pl.BlockSpec((B, T, _EMBED), lambda e, t: (0, 0, 0)),
        out_shape=jax.ShapeDtypeStruct((B, T, _EMBED), x.dtype),
        scratch_shapes=[
            pltpu.VMEM((n_tok, _NEXP), jnp.float32),
            pltpu.VMEM((n_tok, _EMBED), jnp.float32),
            pltpu.VMEM((n_tok, _EMBED), jnp.float32),
        ],
        compiler_params=pltpu.CompilerParams(
            dimension_semantics=("arbitrary", "arbitrary"),
        ),
    )(x, Wr, W1, B1, W2, B2)
    return out
